# baseline (speedup 1.0000x reference)
import functools
import math

import jax
import jax.numpy as jnp
from jax import lax
from jax.experimental import pallas as pl
from jax.experimental.pallas import tpu as pltpu
from jax.experimental.pallas import tpu_sc as plsc

EMB = 64
SCALE = float(math.sqrt(EMB))
NC, NS = 2, 16
NW = NC * NS
G = 128
TBLK = 32768


def _transpose_scale_pad(tT):
    V = tT.shape[1]
    P = jnp.concatenate(
        [jnp.eye(EMB, dtype=jnp.bfloat16) * jnp.bfloat16(SCALE),
         jnp.zeros((EMB, EMB), jnp.bfloat16)], axis=1)

    def body(x_ref, p_ref, o_ref):
        x = x_ref[...]
        p = p_ref[...]
        xhi = x.astype(jnp.bfloat16)
        xlo = (x - xhi.astype(jnp.float32)).astype(jnp.bfloat16)
        dims = (((0,), (0,)), ((), ()))
        o_ref[...] = (
            jax.lax.dot_general(xhi, p, dims,
                                preferred_element_type=jnp.float32)
            + jax.lax.dot_general(xlo, p, dims,
                                  preferred_element_type=jnp.float32)
        )

    return pl.pallas_call(
        body,
        grid=(pl.cdiv(V, TBLK),),
        in_specs=[pl.BlockSpec((EMB, TBLK), lambda i: (0, i)),
                  pl.BlockSpec((EMB, 2 * EMB), lambda i: (0, 0))],
        out_specs=pl.BlockSpec((TBLK, 2 * EMB), lambda i: (i, 0)),
        out_shape=jax.ShapeDtypeStruct((V, 2 * EMB), jnp.float32),
        compiler_params=pltpu.CompilerParams(
            dimension_semantics=("parallel",)),
    )(tT, P)


def kernel(tokens, table):
    B = tokens.shape[0] * tokens.shape[1]
    V = table.shape[0]
    b_per_w = B // NW
    MACRO = 1024
    HALF = 256
    macros = b_per_w // MACRO
    assert b_per_w % MACRO == 0

    table2 = _transpose_scale_pad(jnp.swapaxes(table, 0, 1))
    idx = tokens.reshape(B // G, G).astype(jnp.int32)
    mesh = plsc.VectorSubcoreMesh(core_axis_name="c", subcore_axis_name="s")

    @functools.partial(
        pl.kernel,
        mesh=mesh,
        out_type=jax.ShapeDtypeStruct((B // 2, 2 * EMB), jnp.float32),
        scratch_types=[
            pltpu.VMEM((MACRO // G, G), jnp.int32),
            pltpu.VMEM((HALF, 2 * EMB), jnp.float32),
            pltpu.VMEM((HALF, 2 * EMB), jnp.float32),
            pltpu.VMEM((HALF // 2, 2 * EMB), jnp.float32),
            pltpu.VMEM((HALF // 2, 2 * EMB), jnp.float32),
            pltpu.SemaphoreType.DMA,
            pltpu.SemaphoreType.DMA,
        ],
    )
    def emb_kernel(idx_hbm, table_hbm, out_hbm, idx_v, gath0, gath1,
                   cmp0, cmp1, semA, semB):
        wid = lax.axis_index("s") * NC + lax.axis_index("c")
        base = wid * b_per_w

        @pl.loop(0, macros)
        def _(i):
            off = pl.multiple_of(base + i * MACRO, MACRO)
            row0 = pl.multiple_of(off // G, MACRO // G)
            pltpu.sync_copy(idx_hbm.at[pl.ds(row0, MACRO // G)], idx_v)
            gaths = (gath0, gath1)
            cmps = (cmp0, cmp1)
            sems = (semA, semB)
            nq = MACRO // HALF

            def fire(q):
                for j in range(HALF // G):
                    pltpu.async_copy(
                        table_hbm.at[idx_v.at[q * (HALF // G) + j]],
                        gaths[q % 2].at[pl.ds(j * G, G)],
                        sems[q % 2],
                    )

            def drain_out(q):
                for j in range(HALF // G):
                    pltpu.make_async_copy(
                        table_hbm.at[idx_v.at[q * (HALF // G) + j]],
                        gaths[q % 2].at[pl.ds(j * G, G)],
                        sems[q % 2],
                    ).wait()
                gath, cmp = gaths[q % 2], cmps[q % 2]

                @pl.loop(0, HALF // 2)
                def _(r2):
                    for c in range(0, EMB, 16):
                        cmp[r2, pl.ds(c, 16)] = gath[2 * r2, pl.ds(c, 16)]
                        cmp[r2, pl.ds(EMB + c, 16)] = (
                            gath[2 * r2 + 1, pl.ds(c, 16)])

                pltpu.sync_copy(
                    cmp,
                    out_hbm.at[pl.ds(
                        pl.multiple_of((off + q * HALF) // 2, HALF // 2),
                        HALF // 2)],
                )

            fire(0)
            for q in range(1, nq):
                fire(q)
                drain_out(q - 1)
            drain_out(nq - 1)

    out = emb_kernel(idx, table2)
    return out.reshape(tokens.shape + (EMB,))


# speedup vs baseline: 1.7477x; 1.7477x over previous
import functools
import math

import jax
import jax.numpy as jnp
from jax import lax
from jax.experimental import pallas as pl
from jax.experimental.pallas import tpu as pltpu
from jax.experimental.pallas import tpu_sc as plsc

EMB = 64
SCALE = float(math.sqrt(EMB))
NC, NS = 2, 16
NW = NC * NS
G = 128
TBLK = 32768


def _transpose_scale_pad(tT):
    V = tT.shape[1]
    P = jnp.concatenate(
        [jnp.eye(EMB, dtype=jnp.bfloat16) * jnp.bfloat16(SCALE),
         jnp.zeros((EMB, EMB), jnp.bfloat16)], axis=1)

    def body(x_ref, p_ref, o_ref):
        x = x_ref[...]
        p = p_ref[...]
        xhi = x.astype(jnp.bfloat16)
        xlo = (x - xhi.astype(jnp.float32)).astype(jnp.bfloat16)
        dims = (((0,), (0,)), ((), ()))
        o_ref[...] = (
            jax.lax.dot_general(xhi, p, dims,
                                preferred_element_type=jnp.float32)
            + jax.lax.dot_general(xlo, p, dims,
                                  preferred_element_type=jnp.float32)
        )

    return pl.pallas_call(
        body,
        grid=(pl.cdiv(V, TBLK),),
        in_specs=[pl.BlockSpec((EMB, TBLK), lambda i: (0, i)),
                  pl.BlockSpec((EMB, 2 * EMB), lambda i: (0, 0))],
        out_specs=pl.BlockSpec((TBLK, 2 * EMB), lambda i: (i, 0)),
        out_shape=jax.ShapeDtypeStruct((V, 2 * EMB), jnp.float32),
        compiler_params=pltpu.CompilerParams(
            dimension_semantics=("parallel",)),
    )(tT, P)


def kernel(tokens, table):
    B = tokens.shape[0] * tokens.shape[1]
    V = table.shape[0]
    b_per_w = B // NW
    Q = 256
    nq = b_per_w // Q
    assert b_per_w % Q == 0 and nq % 2 == 0

    table2 = _transpose_scale_pad(jnp.swapaxes(table, 0, 1))
    idx = tokens.reshape(B // G, G).astype(jnp.int32)
    mesh = plsc.VectorSubcoreMesh(core_axis_name="c", subcore_axis_name="s")

    @functools.partial(
        pl.kernel,
        mesh=mesh,
        out_type=jax.ShapeDtypeStruct((B, 2 * EMB), jnp.float32),
        scratch_types=[
            pltpu.VMEM((b_per_w // G, G), jnp.int32),
            pltpu.VMEM((Q, 2 * EMB), jnp.float32),
            pltpu.VMEM((Q, 2 * EMB), jnp.float32),
            pltpu.SemaphoreType.DMA,
            pltpu.SemaphoreType.DMA,
        ],
    )
    def emb_kernel(idx_hbm, table_hbm, out_hbm, idx_v, gath0, gath1,
                   semA, semB):
        wid = lax.axis_index("s") * NC + lax.axis_index("c")
        base = wid * b_per_w
        pltpu.sync_copy(
            idx_hbm.at[pl.ds(pl.multiple_of(base // G, b_per_w // G),
                             b_per_w // G)],
            idx_v)
        gaths = (gath0, gath1)
        sems = (semA, semB)

        def fire(q, b):
            for j in range(Q // G):
                pltpu.async_copy(
                    table_hbm.at[idx_v.at[q * (Q // G) + j]],
                    gaths[b].at[pl.ds(j * G, G)],
                    sems[b],
                )

        def drain_out(q, b):
            for j in range(Q // G):
                pltpu.make_async_copy(
                    table_hbm.at[idx_v.at[q * (Q // G) + j]],
                    gaths[b].at[pl.ds(j * G, G)],
                    sems[b],
                ).wait()
            pltpu.sync_copy(
                gaths[b],
                out_hbm.at[pl.ds(pl.multiple_of(base + q * Q, Q), Q)],
            )

        fire(0, 0)

        @pl.loop(0, nq // 2)
        def _(i):
            q = i * 2
            fire(q + 1, 1)
            drain_out(q, 0)

            @pl.when(i < nq // 2 - 1)
            def _():
                fire(q + 2, 0)

            drain_out(q + 1, 1)

    out = emb_kernel(idx, table2)
    return out[:, :EMB].reshape(tokens.shape + (EMB,))
